# baseline (speedup 1.0000x reference)
import functools

import jax
import jax.numpy as jnp
from jax import lax
from jax.experimental import pallas as pl
from jax.experimental.pallas import tpu as pltpu
from jax.experimental.pallas import tpu_sc as plsc

B = 16384
DIM = 32
NCH = 8
SIZE = 16777216
LANES = 16
TROWS = SIZE // LANES
RMASK = TROWS - 1
PRIME = (1 << 31) - 1

NC, NS = 2, 16
NW = NC * NS
RPW = B // NW
NB = 16
NBLK = RPW // NB
LPB = NB * NCH


def _fold(s):
    return (s & jnp.uint32(PRIME)) + (s >> 31)


def _rot(n, k):
    low = (n & jnp.uint32((1 << (31 - k)) - 1)) << k
    high = n >> (31 - k)
    return low + high


def _hash(x1, x0, a1, a0, bb):
    s = _fold(_rot(a1 * x1, 26) + a0 * x0)
    s = _fold(s + _rot(a1 * x0, 16))
    s = _fold(s + _rot(a0 * x1, 10))
    s = _fold(s + bb)
    s = _fold(s)
    s = jnp.where(s >= jnp.uint32(PRIME), s - jnp.uint32(PRIME), s)
    return s & jnp.uint32(SIZE - 1)


def _body(x_hbm, tab_hbm, cf_hbm, out_hbm,
          xv, cfv, sidxA, sidxB, widxA, widxB, o0A, o0B, o1A, o1B,
          sstageA, sstageB, wstageA, wstageB, outv, semA, semB):
    wid = lax.axis_index("s") * jnp.int32(NC) + lax.axis_index("c")
    base = wid * jnp.int32(RPW)
    pltpu.sync_copy(x_hbm.at[pl.ds(base, RPW)], xv)
    pltpu.sync_copy(cf_hbm, cfv)
    lanes = lax.iota(jnp.int32, LANES)
    zero = lanes * 0
    j8 = lanes & 7
    i0, i1, i2 = jnp.int32(0), jnp.int32(1), jnp.int32(2)

    cfr = [cfv[pl.ds(r * LANES, LANES)] for r in range(6)]
    coef = [[cfr[r][j] for r in range(6)] for j in range(NCH)]

    def hash_block(b, sidx, widx, o0b, o1b):
        xu = xv[pl.ds(b * jnp.int32(NB), NB)]
        x1 = xu >> 10
        x0 = xu & jnp.uint32(1023)
        for j in range(NCH):
            a1_0, a0_0, b_0, a1_1, a0_1, b_1 = coef[j]
            h0 = _hash(x1, x0, a1_0, a0_0, b_0)
            r = (h0 >> 4).astype(jnp.int32)
            o0 = (h0 & jnp.uint32(15)).astype(jnp.int32)
            h1 = _hash(x1, x0, a1_1, a0_1, b_1)
            wr = (h1 >> 4).astype(jnp.int32)
            wo = (h1 & jnp.uint32(15)).astype(jnp.int32)
            lkv = lanes * NCH + j
            plsc.store_scatter(sidx, [lkv], r)
            plsc.store_scatter(sidx, [lkv + LPB], (r + 1) & RMASK)
            plsc.store_scatter(sidx, [lkv + 2 * LPB], (r + 2) & RMASK)
            plsc.store_scatter(widx, [lkv], wr)
            plsc.store_scatter(o0b, [lanes * LANES + j], o0)
            plsc.store_scatter(o1b, [lanes * LANES + j], wo)

    def copies(sidx, widx, sstage, wstage, sem):
        return (
            pltpu.make_async_copy(tab_hbm.at[sidx.at[pl.ds(0, LPB)]],
                                  sstage.at[i0], sem),
            pltpu.make_async_copy(tab_hbm.at[sidx.at[pl.ds(LPB, LPB)]],
                                  sstage.at[i1], sem),
            pltpu.make_async_copy(tab_hbm.at[sidx.at[pl.ds(2 * LPB, LPB)]],
                                  sstage.at[i2], sem),
            pltpu.make_async_copy(tab_hbm.at[widx], wstage, sem),
        )

    def fire(bufs):
        for c in copies(*bufs):
            c.start()

    def drain(bufs):
        for c in copies(*bufs):
            c.wait()

    def accum(b, o0b, o1b, sstage, wstage):
        for row in range(NB):
            ov = o0b[pl.ds(row * LANES, LANES)]
            o1v = o1b[pl.ds(row * LANES, LANES)]
            wv = plsc.load_gather(wstage, [row * NCH + j8, o1v & 15])
            acc0 = lanes * jnp.float32(0.0)
            acc1 = lanes * jnp.float32(0.0)
            for j in range(NCH):
                lk = row * NCH + j
                p0 = ov[j] + lanes
                p1 = p0 + 16
                lkf = zero + lk
                g0 = plsc.load_gather(sstage, [p0 >> 4, lkf, p0 & 15])
                g1 = plsc.load_gather(sstage, [p1 >> 4, lkf, p1 & 15])
                w = wv[j]
                acc0 = acc0 + g0 * w
                acc1 = acc1 + g1 * w
            orow = b * jnp.int32(NB) + row
            outv[orow, pl.ds(0, LANES)] = acc0 * 2.0
            outv[orow, pl.ds(LANES, LANES)] = acc1 * 2.0

    bufsA = (sidxA, widxA, sstageA, wstageA, semA)
    bufsB = (sidxB, widxB, sstageB, wstageB, semB)

    hash_block(jnp.int32(0), sidxA, widxA, o0A, o1A)
    fire(bufsA)

    def pair(i, carry):
        b0 = i * jnp.int32(2)
        b1 = b0 + 1
        hash_block(b1, sidxB, widxB, o0B, o1B)
        fire(bufsB)
        drain(bufsA)
        accum(b0, o0A, o1A, sstageA, wstageA)

        @pl.when(i < jnp.int32(NBLK // 2 - 1))
        def _():
            hash_block(b0 + 2, sidxA, widxA, o0A, o1A)
            fire(bufsA)

        drain(bufsB)
        accum(b1, o0B, o1B, sstageB, wstageB)
        return carry

    lax.fori_loop(jnp.int32(0), jnp.int32(NBLK // 2), pair, jnp.int32(0))
    pltpu.sync_copy(outv, out_hbm.at[pl.ds(base, RPW)])


@jax.jit
def _sc_call(xs, tab2d, cf):
    mesh = plsc.VectorSubcoreMesh(core_axis_name="c", subcore_axis_name="s")
    f = functools.partial(
        pl.kernel,
        out_type=jax.ShapeDtypeStruct((B, DIM), jnp.float32),
        mesh=mesh,
        scratch_types=[
            pltpu.VMEM((RPW,), jnp.uint32),
            pltpu.VMEM((6 * LANES,), jnp.uint32),
            pltpu.VMEM((3 * LPB,), jnp.int32),
            pltpu.VMEM((3 * LPB,), jnp.int32),
            pltpu.VMEM((LPB,), jnp.int32),
            pltpu.VMEM((LPB,), jnp.int32),
            pltpu.VMEM((NB * LANES,), jnp.int32),
            pltpu.VMEM((NB * LANES,), jnp.int32),
            pltpu.VMEM((NB * LANES,), jnp.int32),
            pltpu.VMEM((NB * LANES,), jnp.int32),
            pltpu.VMEM((3, LPB, LANES), jnp.float32),
            pltpu.VMEM((3, LPB, LANES), jnp.float32),
            pltpu.VMEM((LPB, LANES), jnp.float32),
            pltpu.VMEM((LPB, LANES), jnp.float32),
            pltpu.VMEM((RPW, DIM), jnp.float32),
            pltpu.SemaphoreType.DMA,
            pltpu.SemaphoreType.DMA,
        ],
        compiler_params=pltpu.CompilerParams(
            needs_layout_passes=False, use_tc_tiling_on_sc=False),
    )(_body)
    return f(xs, tab2d, cf)


def kernel(x, table0, coeffs0, coeffs1):
    xs = x.astype(jnp.uint32)
    tab2d = table0.reshape(TROWS, LANES)

    def split(c):
        a = c[:, 0]
        return jnp.stack([a >> 16, a & 0xFFFF, c[:, 1]])

    cf = jnp.concatenate([split(coeffs0), split(coeffs1)]).astype(jnp.uint32)
    cf = jnp.pad(cf, ((0, 0), (0, LANES - NCH))).reshape(-1)
    return _sc_call(xs, tab2d, cf)


# speedup vs baseline: 1.1601x; 1.1601x over previous
import functools

import jax
import jax.numpy as jnp
from jax import lax
from jax.experimental import pallas as pl
from jax.experimental.pallas import tpu as pltpu
from jax.experimental.pallas import tpu_sc as plsc

B = 16384
DIM = 32
NCH = 8
SIZE = 16777216
LANES = 16
TROWS = SIZE // LANES
RMASK = TROWS - 1
PRIME = (1 << 31) - 1

NC, NS = 2, 16
NW = NC * NS
RPW = B // NW
NB = 16
NBLK = RPW // NB
LPB = NB * NCH


def _fold(s):
    return (s & jnp.uint32(PRIME)) + (s >> 31)


def _rot(n, k):
    low = (n & jnp.uint32((1 << (31 - k)) - 1)) << k
    high = n >> (31 - k)
    return low + high


def _hash(x1, x0, a1, a0, bb):
    s = _fold(_rot(a1 * x1, 26) + a0 * x0)
    s = _fold(s + _rot(a1 * x0, 16))
    s = _fold(s + _rot(a0 * x1, 10))
    s = _fold(s + bb)
    s = _fold(s)
    s = jnp.where(s >= jnp.uint32(PRIME), s - jnp.uint32(PRIME), s)
    return s & jnp.uint32(SIZE - 1)


SROWS = 4 * LPB
SFLAT = SROWS * LANES


def _body(x_hbm, tab_hbm, cf_hbm, out_hbm,
          xv, cfv, sidxA, sidxB, bvA, bvB, wbA, wbB,
          sstageA, sstageB, outv, semA, semB):
    wid = lax.axis_index("s") * jnp.int32(NC) + lax.axis_index("c")
    base = wid * jnp.int32(RPW)
    pltpu.sync_copy(x_hbm.at[pl.ds(base, RPW)], xv)
    pltpu.sync_copy(cf_hbm, cfv)
    lanes = lax.iota(jnp.int32, LANES)
    lanes512 = lanes * 512

    cfr = [cfv[pl.ds(r * LANES, LANES)] for r in range(6)]
    coef = [[cfr[r][j] for r in range(6)] for j in range(NCH)]

    def hash_block(b, sidx, bv, wb):
        xu = xv[pl.ds(b * jnp.int32(NB), NB)]
        x1 = xu >> 10
        x0 = xu & jnp.uint32(1023)
        for j in range(NCH):
            a1_0, a0_0, b_0, a1_1, a0_1, b_1 = coef[j]
            h0 = _hash(x1, x0, a1_0, a0_0, b_0)
            r = (h0 >> 4).astype(jnp.int32)
            o0 = (h0 & jnp.uint32(15)).astype(jnp.int32)
            h1 = _hash(x1, x0, a1_1, a0_1, b_1)
            wr = (h1 >> 4).astype(jnp.int32)
            wo = (h1 & jnp.uint32(15)).astype(jnp.int32)
            posb = lanes * 32 + 4 * j
            plsc.store_scatter(sidx, [posb], r)
            plsc.store_scatter(sidx, [posb + 1], (r + 1) & RMASK)
            plsc.store_scatter(sidx, [posb + 2], (r + 2) & RMASK)
            plsc.store_scatter(sidx, [posb + 3], wr)
            slot = lanes * LANES + j
            plsc.store_scatter(bv, [slot], lanes512 + (o0 + 64 * j))
            plsc.store_scatter(wb, [slot], lanes512 + (wo + (64 * j + 48)))

    def copies(sidx, sstage, sem):
        return [
            pltpu.make_async_copy(
                tab_hbm.at[sidx.at[pl.ds(d * LPB, LPB)]],
                sstage.at[pl.ds(d * LPB, LPB)], sem)
            for d in range(4)
        ]

    def fire(bufs):
        for c in copies(*bufs):
            c.start()

    def drain(bufs):
        for c in copies(*bufs):
            c.wait()

    def accum(b, bv, wb, sstage):
        for row in range(NB):
            bvrow = bv[pl.ds(row * LANES, LANES)]
            wposv = wb[pl.ds(row * LANES, LANES)]
            wp = wposv & (SFLAT - 1)
            wv = plsc.load_gather(sstage, [wp >> 4, wp & 15])
            prods0 = []
            prods1 = []
            for j in range(NCH):
                f0 = bvrow[j] + lanes
                f1 = f0 + 16
                g0 = plsc.load_gather(sstage, [f0 >> 4, f0 & 15])
                g1 = plsc.load_gather(sstage, [f1 >> 4, f1 & 15])
                w = wv[j]
                prods0.append(g0 * w)
                prods1.append(g1 * w)
            acc0 = ((prods0[0] + prods0[1]) + (prods0[2] + prods0[3])) + (
                (prods0[4] + prods0[5]) + (prods0[6] + prods0[7]))
            acc1 = ((prods1[0] + prods1[1]) + (prods1[2] + prods1[3])) + (
                (prods1[4] + prods1[5]) + (prods1[6] + prods1[7]))
            orow = b * jnp.int32(NB) + row
            outv[orow, pl.ds(0, LANES)] = acc0 * 2.0
            outv[orow, pl.ds(LANES, LANES)] = acc1 * 2.0

    bufsA = (sidxA, sstageA, semA)
    bufsB = (sidxB, sstageB, semB)

    hash_block(jnp.int32(0), sidxA, bvA, wbA)
    fire(bufsA)

    def pair(i, carry):
        b0 = i * jnp.int32(2)
        b1 = b0 + 1
        hash_block(b1, sidxB, bvB, wbB)
        fire(bufsB)
        drain(bufsA)
        accum(b0, bvA, wbA, sstageA)

        @pl.when(i < jnp.int32(NBLK // 2 - 1))
        def _():
            hash_block(b0 + 2, sidxA, bvA, wbA)
            fire(bufsA)

        drain(bufsB)
        accum(b1, bvB, wbB, sstageB)
        return carry

    lax.fori_loop(jnp.int32(0), jnp.int32(NBLK // 2), pair, jnp.int32(0))
    pltpu.sync_copy(outv, out_hbm.at[pl.ds(base, RPW)])


@jax.jit
def _sc_call(xs, tab2d, cf):
    mesh = plsc.VectorSubcoreMesh(core_axis_name="c", subcore_axis_name="s")
    f = functools.partial(
        pl.kernel,
        out_type=jax.ShapeDtypeStruct((B, DIM), jnp.float32),
        mesh=mesh,
        scratch_types=[
            pltpu.VMEM((RPW,), jnp.uint32),
            pltpu.VMEM((6 * LANES,), jnp.uint32),
            pltpu.VMEM((4 * LPB,), jnp.int32),
            pltpu.VMEM((4 * LPB,), jnp.int32),
            pltpu.VMEM((NB * LANES,), jnp.int32),
            pltpu.VMEM((NB * LANES,), jnp.int32),
            pltpu.VMEM((NB * LANES,), jnp.int32),
            pltpu.VMEM((NB * LANES,), jnp.int32),
            pltpu.VMEM((SROWS, LANES), jnp.float32),
            pltpu.VMEM((SROWS, LANES), jnp.float32),
            pltpu.VMEM((RPW, DIM), jnp.float32),
            pltpu.SemaphoreType.DMA,
            pltpu.SemaphoreType.DMA,
        ],
        compiler_params=pltpu.CompilerParams(
            needs_layout_passes=False, use_tc_tiling_on_sc=False),
    )(_body)
    return f(xs, tab2d, cf)


def kernel(x, table0, coeffs0, coeffs1):
    xs = x.astype(jnp.uint32)
    tab2d = table0.reshape(TROWS, LANES)

    def split(c):
        a = c[:, 0]
        return jnp.stack([a >> 16, a & 0xFFFF, c[:, 1]])

    cf = jnp.concatenate([split(coeffs0), split(coeffs1)]).astype(jnp.uint32)
    cf = jnp.pad(cf, ((0, 0), (0, LANES - NCH))).reshape(-1)
    return _sc_call(xs, tab2d, cf)


# speedup vs baseline: 1.2988x; 1.1195x over previous
import functools

import jax
import jax.numpy as jnp
from jax import lax
from jax.experimental import pallas as pl
from jax.experimental.pallas import tpu as pltpu
from jax.experimental.pallas import tpu_sc as plsc

B = 16384
DIM = 32
NCH = 8
SIZE = 16777216
LANES = 16
TROWS = SIZE // LANES
RMASK = TROWS - 1
PRIME = (1 << 31) - 1

NC, NS = 2, 16
NW = NC * NS
RPW = B // NW
NB = 16
NBLK = RPW // NB
LPB = NB * NCH


def _fold(s):
    return (s & jnp.uint32(PRIME)) + (s >> 31)


def _rot(n, k):
    low = (n & jnp.uint32((1 << (31 - k)) - 1)) << k
    high = n >> (31 - k)
    return low + high


def _hash(x1, x0, a1, a0, bb):
    s = _fold(_rot(a1 * x1, 26) + a0 * x0)
    s = _fold(s + _rot(a1 * x0, 16))
    s = _fold(s + _rot(a0 * x1, 10))
    s = _fold(s + bb)
    s = _fold(s)
    s = jnp.where(s >= jnp.uint32(PRIME), s - jnp.uint32(PRIME), s)
    return s & jnp.uint32(SIZE - 1)


SROWS = 4 * LPB
SFLAT = SROWS * LANES


def _body(x_hbm, tab_hbm, cf_hbm, out_hbm,
          xv, cfv, sidxA, sidxB, bvA, bvB, wbA, wbB,
          sstageA, sstageB, outv, semA, semB):
    wid = lax.axis_index("s") * jnp.int32(NC) + lax.axis_index("c")
    base = wid * jnp.int32(RPW)
    pltpu.sync_copy(x_hbm.at[pl.ds(base, RPW)], xv)
    pltpu.sync_copy(cf_hbm, cfv)
    lanes = lax.iota(jnp.int32, LANES)
    lanes512 = lanes * 512

    cfr = [cfv[pl.ds(r * LANES, LANES)] for r in range(6)]
    coef = [[cfr[r][j] for r in range(6)] for j in range(NCH)]

    def hash_block(b, sidx, bv, wb):
        xu = xv[pl.ds(b * jnp.int32(NB), NB)]
        x1 = xu >> 10
        x0 = xu & jnp.uint32(1023)
        for j in range(NCH):
            a1_0, a0_0, b_0, a1_1, a0_1, b_1 = coef[j]
            h0 = _hash(x1, x0, a1_0, a0_0, b_0)
            r = (h0 >> 4).astype(jnp.int32)
            o0 = (h0 & jnp.uint32(15)).astype(jnp.int32)
            h1 = _hash(x1, x0, a1_1, a0_1, b_1)
            wr = (h1 >> 4).astype(jnp.int32)
            wo = (h1 & jnp.uint32(15)).astype(jnp.int32)
            posb = lanes * 32 + 4 * j
            plsc.store_scatter(sidx, [posb], r)
            plsc.store_scatter(sidx, [posb + 1], (r + 1) & RMASK)
            plsc.store_scatter(sidx, [posb + 2], (r + 2) & RMASK)
            plsc.store_scatter(sidx, [posb + 3], wr)
            slot = lanes * LANES + j
            plsc.store_scatter(bv, [slot], lanes512 + (o0 + 64 * j))
            plsc.store_scatter(wb, [slot], lanes512 + (wo + (64 * j + 48)))

    def copies(sidx, sstage, sem):
        return [
            pltpu.make_async_copy(
                tab_hbm.at[sidx.at[pl.ds(d * LPB, LPB)]],
                sstage.at[pl.ds(d * LPB, LPB)], sem)
            for d in range(4)
        ]

    def fire(bufs):
        for c in copies(*bufs):
            c.start()

    def drain(bufs):
        for c in copies(*bufs):
            c.wait()

    def _tree8(p):
        return ((p[0] + p[1]) + (p[2] + p[3])) + ((p[4] + p[5]) + (p[6] + p[7]))

    def accum(b, bv, wb, sstage):
        for rp in range(NB // 2):
            rows = (2 * rp, 2 * rp + 1)
            bvr, wvr, prods = [], [], []
            for row in rows:
                bvrow = bv[pl.ds(row * LANES, LANES)]
                wposv = wb[pl.ds(row * LANES, LANES)]
                wp = wposv & (SFLAT - 1)
                bvr.append(bvrow)
                wvr.append(plsc.load_gather(sstage, [wp >> 4, wp & 15]))
                prods.append(([], []))
            for j in range(NCH):
                for t in range(2):
                    f0 = bvr[t][j] + lanes
                    f1 = f0 + 16
                    g0 = plsc.load_gather(sstage, [f0 >> 4, f0 & 15])
                    g1 = plsc.load_gather(sstage, [f1 >> 4, f1 & 15])
                    w = wvr[t][j]
                    prods[t][0].append(g0 * w)
                    prods[t][1].append(g1 * w)
            for t in range(2):
                orow = b * jnp.int32(NB) + rows[t]
                outv[orow, pl.ds(0, LANES)] = _tree8(prods[t][0]) * 2.0
                outv[orow, pl.ds(LANES, LANES)] = _tree8(prods[t][1]) * 2.0

    bufsA = (sidxA, sstageA, semA)
    bufsB = (sidxB, sstageB, semB)

    hash_block(jnp.int32(0), sidxA, bvA, wbA)
    fire(bufsA)

    def pair(i, carry):
        b0 = i * jnp.int32(2)
        b1 = b0 + 1
        hash_block(b1, sidxB, bvB, wbB)
        fire(bufsB)
        drain(bufsA)
        accum(b0, bvA, wbA, sstageA)

        @pl.when(i < jnp.int32(NBLK // 2 - 1))
        def _():
            hash_block(b0 + 2, sidxA, bvA, wbA)
            fire(bufsA)

        drain(bufsB)
        accum(b1, bvB, wbB, sstageB)
        return carry

    lax.fori_loop(jnp.int32(0), jnp.int32(NBLK // 2), pair, jnp.int32(0))
    pltpu.sync_copy(outv, out_hbm.at[pl.ds(base, RPW)])


@jax.jit
def _sc_call(xs, tab2d, cf):
    mesh = plsc.VectorSubcoreMesh(core_axis_name="c", subcore_axis_name="s")
    f = functools.partial(
        pl.kernel,
        out_type=jax.ShapeDtypeStruct((B, DIM), jnp.float32),
        mesh=mesh,
        scratch_types=[
            pltpu.VMEM((RPW,), jnp.uint32),
            pltpu.VMEM((6 * LANES,), jnp.uint32),
            pltpu.VMEM((4 * LPB,), jnp.int32),
            pltpu.VMEM((4 * LPB,), jnp.int32),
            pltpu.VMEM((NB * LANES,), jnp.int32),
            pltpu.VMEM((NB * LANES,), jnp.int32),
            pltpu.VMEM((NB * LANES,), jnp.int32),
            pltpu.VMEM((NB * LANES,), jnp.int32),
            pltpu.VMEM((SROWS, LANES), jnp.float32),
            pltpu.VMEM((SROWS, LANES), jnp.float32),
            pltpu.VMEM((RPW, DIM), jnp.float32),
            pltpu.SemaphoreType.DMA,
            pltpu.SemaphoreType.DMA,
        ],
        compiler_params=pltpu.CompilerParams(
            needs_layout_passes=False, use_tc_tiling_on_sc=False),
    )(_body)
    return f(xs, tab2d, cf)


def kernel(x, table0, coeffs0, coeffs1):
    xs = x.astype(jnp.uint32)
    tab2d = table0.reshape(TROWS, LANES)

    def split(c):
        a = c[:, 0]
        return jnp.stack([a >> 16, a & 0xFFFF, c[:, 1]])

    cf = jnp.concatenate([split(coeffs0), split(coeffs1)]).astype(jnp.uint32)
    cf = jnp.pad(cf, ((0, 0), (0, LANES - NCH))).reshape(-1)
    return _sc_call(xs, tab2d, cf)


# speedup vs baseline: 1.2992x; 1.0004x over previous
import functools

import jax
import jax.numpy as jnp
from jax import lax
from jax.experimental import pallas as pl
from jax.experimental.pallas import tpu as pltpu
from jax.experimental.pallas import tpu_sc as plsc

B = 16384
DIM = 32
NCH = 8
SIZE = 16777216
LANES = 16
TROWS = SIZE // LANES
RMASK = TROWS - 1
PRIME = (1 << 31) - 1

NC, NS = 2, 16
NW = NC * NS
RPW = B // NW
NB = 16
NBLK = RPW // NB
LPB = NB * NCH


def _fold(s):
    return (s & jnp.uint32(PRIME)) + (s >> 31)


def _rot(n, k):
    low = (n & jnp.uint32((1 << (31 - k)) - 1)) << k
    high = n >> (31 - k)
    return low + high


def _hash(x1, x0, a1, a0, bb):
    s = _fold(_rot(a1 * x1, 26) + a0 * x0)
    s = _fold(s + _rot(a1 * x0, 16))
    s = _fold(s + _rot(a0 * x1, 10))
    s = _fold(s + bb)
    s = _fold(s)
    s = jnp.where(s >= jnp.uint32(PRIME), s - jnp.uint32(PRIME), s)
    return s & jnp.uint32(SIZE - 1)


SROWS = 4 * LPB
SFLAT = SROWS * LANES


def _body(x_hbm, tab_hbm, cf_hbm, out_hbm,
          xv, cfv, sidxA, sidxB, bvA, bvB, wbA, wbB,
          sstageA, sstageB, outv, semA, semB):
    wid = lax.axis_index("s") * jnp.int32(NC) + lax.axis_index("c")
    base = wid * jnp.int32(RPW)
    pltpu.sync_copy(x_hbm.at[pl.ds(base, RPW)], xv)
    pltpu.sync_copy(cf_hbm, cfv)
    lanes = lax.iota(jnp.int32, LANES)
    lanes512 = lanes * 512

    cfr = [cfv[pl.ds(r * LANES, LANES)] for r in range(6)]
    coef = [[cfr[r][j] for r in range(6)] for j in range(NCH)]

    def hash_block(b, sidx, bv, wb):
        xu = xv[pl.ds(b * jnp.int32(NB), NB)]
        x1 = xu >> 10
        x0 = xu & jnp.uint32(1023)
        for j in range(NCH):
            a1_0, a0_0, b_0, a1_1, a0_1, b_1 = coef[j]
            h0 = _hash(x1, x0, a1_0, a0_0, b_0)
            r = (h0 >> 4).astype(jnp.int32)
            o0 = (h0 & jnp.uint32(15)).astype(jnp.int32)
            h1 = _hash(x1, x0, a1_1, a0_1, b_1)
            wr = (h1 >> 4).astype(jnp.int32)
            wo = (h1 & jnp.uint32(15)).astype(jnp.int32)
            posb = lanes * 32 + 4 * j
            plsc.store_scatter(sidx, [posb], r)
            plsc.store_scatter(sidx, [posb + 1], (r + 1) & RMASK)
            plsc.store_scatter(sidx, [posb + 2], (r + 2) & RMASK)
            plsc.store_scatter(sidx, [posb + 3], wr)
            slot = lanes * LANES + j
            plsc.store_scatter(bv, [slot], lanes512 + (o0 + 64 * j))
            plsc.store_scatter(wb, [slot], lanes512 + (wo + (64 * j + 48)))

    def copies(sidx, sstage, sem):
        return [
            pltpu.make_async_copy(
                tab_hbm.at[sidx.at[pl.ds(d * LPB, LPB)]],
                sstage.at[pl.ds(d * LPB, LPB)], sem)
            for d in range(4)
        ]

    def fire(bufs):
        for c in copies(*bufs):
            c.start()

    def drain(bufs):
        for c in copies(*bufs):
            c.wait()

    def _tree8(p):
        return ((p[0] + p[1]) + (p[2] + p[3])) + ((p[4] + p[5]) + (p[6] + p[7]))

    def accum(b, bv, wb, sstage):
        for rp in range(NB // 2):
            rows = (2 * rp, 2 * rp + 1)
            bvr, wvr, prods = [], [], []
            for row in rows:
                bvrow = bv[pl.ds(row * LANES, LANES)]
                wposv = wb[pl.ds(row * LANES, LANES)]
                wp = wposv & (SFLAT - 1)
                bvr.append(bvrow)
                wvr.append(plsc.load_gather(sstage, [wp >> 4, wp & 15]))
                prods.append(([], []))
            for j in range(NCH):
                for t in range(2):
                    f0 = bvr[t][j] + lanes
                    r0 = f0 >> 4
                    e0 = f0 & 15
                    g0 = plsc.load_gather(sstage, [r0, e0])
                    g1 = plsc.load_gather(sstage, [r0 + 1, e0])
                    w = wvr[t][j]
                    prods[t][0].append(g0 * w)
                    prods[t][1].append(g1 * w)
            for t in range(2):
                orow = b * jnp.int32(NB) + rows[t]
                outv[orow, pl.ds(0, LANES)] = _tree8(prods[t][0]) * 2.0
                outv[orow, pl.ds(LANES, LANES)] = _tree8(prods[t][1]) * 2.0

    bufsA = (sidxA, sstageA, semA)
    bufsB = (sidxB, sstageB, semB)

    hash_block(jnp.int32(0), sidxA, bvA, wbA)
    fire(bufsA)

    def pair(i, carry):
        b0 = i * jnp.int32(2)
        b1 = b0 + 1
        hash_block(b1, sidxB, bvB, wbB)
        fire(bufsB)
        drain(bufsA)
        accum(b0, bvA, wbA, sstageA)

        @pl.when(i < jnp.int32(NBLK // 2 - 1))
        def _():
            hash_block(b0 + 2, sidxA, bvA, wbA)
            fire(bufsA)

        drain(bufsB)
        accum(b1, bvB, wbB, sstageB)
        return carry

    lax.fori_loop(jnp.int32(0), jnp.int32(NBLK // 2), pair, jnp.int32(0))
    pltpu.sync_copy(outv, out_hbm.at[pl.ds(base, RPW)])


@jax.jit
def _sc_call(xs, tab2d, cf):
    mesh = plsc.VectorSubcoreMesh(core_axis_name="c", subcore_axis_name="s")
    f = functools.partial(
        pl.kernel,
        out_type=jax.ShapeDtypeStruct((B, DIM), jnp.float32),
        mesh=mesh,
        scratch_types=[
            pltpu.VMEM((RPW,), jnp.uint32),
            pltpu.VMEM((6 * LANES,), jnp.uint32),
            pltpu.VMEM((4 * LPB,), jnp.int32),
            pltpu.VMEM((4 * LPB,), jnp.int32),
            pltpu.VMEM((NB * LANES,), jnp.int32),
            pltpu.VMEM((NB * LANES,), jnp.int32),
            pltpu.VMEM((NB * LANES,), jnp.int32),
            pltpu.VMEM((NB * LANES,), jnp.int32),
            pltpu.VMEM((SROWS, LANES), jnp.float32),
            pltpu.VMEM((SROWS, LANES), jnp.float32),
            pltpu.VMEM((RPW, DIM), jnp.float32),
            pltpu.SemaphoreType.DMA,
            pltpu.SemaphoreType.DMA,
        ],
        compiler_params=pltpu.CompilerParams(
            needs_layout_passes=False, use_tc_tiling_on_sc=False),
    )(_body)
    return f(xs, tab2d, cf)


def kernel(x, table0, coeffs0, coeffs1):
    xs = x.astype(jnp.uint32)
    tab2d = table0.reshape(TROWS, LANES)

    def split(c):
        a = c[:, 0]
        return jnp.stack([a >> 16, a & 0xFFFF, c[:, 1]])

    cf = jnp.concatenate([split(coeffs0), split(coeffs1)]).astype(jnp.uint32)
    cf = jnp.pad(cf, ((0, 0), (0, LANES - NCH))).reshape(-1)
    return _sc_call(xs, tab2d, cf)
